# baseline (speedup 1.0000x reference)
import jax
import jax.numpy as jnp
from jax.experimental import pallas as pl
from jax.experimental.pallas import tpu as pltpu

N = 4096
F = 256
BM = 512
M_BLOCKS = N // BM
NB = 3
CC = 8
KC = N // CC


def _mm(a, b):
    return jax.lax.dot_general(
        a, b, (((1,), (0,)), ((), ())),
        precision=jax.lax.Precision.DEFAULT,
        preferred_element_type=jnp.float32,
    )


def _feat_mm_kernel(feat_ref, w_ref, y_ref):
    y_ref[...] = _mm(feat_ref[...], w_ref[...])


def _feat_mm(feat, w):
    return pl.pallas_call(
        _feat_mm_kernel,
        out_shape=jax.ShapeDtypeStruct((N, F), jnp.float32),
    )(feat, w)


def _start_block_copies(adj_hbm, buf_ref, sem, block, slot):
    for j in range(CC):
        pltpu.make_async_copy(
            adj_hbm.at[pl.ds(block * BM, BM), pl.ds(j * KC, KC)],
            buf_ref.at[slot, :, pl.ds(j * KC, KC)],
            sem.at[slot, j],
        ).start()


def _wait_block_copies(adj_hbm, buf_ref, sem, block, slot):
    for j in range(CC):
        pltpu.make_async_copy(
            adj_hbm.at[pl.ds(block * BM, BM), pl.ds(j * KC, KC)],
            buf_ref.at[slot, :, pl.ds(j * KC, KC)],
            sem.at[slot, j],
        ).wait()


def _stream_step(adj_hbm, buf_ref, sem):
    m = pl.program_id(0)

    @pl.when(m == 0)
    def _():
        for r in range(NB):
            if r < M_BLOCKS:
                _start_block_copies(adj_hbm, buf_ref, sem, r, r)

    slot = jax.lax.rem(m, NB)
    _wait_block_copies(adj_hbm, buf_ref, sem, m, slot)
    return slot


def _refill_step(adj_hbm, buf_ref, sem):
    m = pl.program_id(0)

    @pl.when(m + NB < M_BLOCKS)
    def _():
        _start_block_copies(adj_hbm, buf_ref, sem, m + NB,
                            jax.lax.rem(m, NB))


def _layer_kernel(adj_hbm, y_ref, b_ref, out_ref, buf_ref, sem):
    slot = _stream_step(adj_hbm, buf_ref, sem)
    acc = _mm(buf_ref[slot], y_ref[...])
    out_ref[...] = jnp.maximum(acc + b_ref[...], 0.0)
    _refill_step(adj_hbm, buf_ref, sem)


def _gcn_layer(adj, y, b):
    return pl.pallas_call(
        _layer_kernel,
        grid=(M_BLOCKS,),
        in_specs=[
            pl.BlockSpec(memory_space=pltpu.MemorySpace.HBM),
            pl.BlockSpec((N, F), lambda m: (0, 0)),
            pl.BlockSpec((1, F), lambda m: (0, 0)),
        ],
        out_specs=pl.BlockSpec((BM, F), lambda m: (m, 0)),
        out_shape=jax.ShapeDtypeStruct((N, F), jnp.float32),
        scratch_shapes=[
            pltpu.VMEM((NB, BM, N), jnp.float32),
            pltpu.SemaphoreType.DMA((NB, CC)),
        ],
    )(adj, y, b)


def _final_kernel(adj_hbm, y_ref, b_ref, wsvm_ref, bsvm_ref, out_ref,
                  buf_ref, sem):
    slot = _stream_step(adj_hbm, buf_ref, sem)
    acc = _mm(buf_ref[slot], y_ref[...])
    h = jnp.maximum(acc + b_ref[...], 0.0)
    logits = _mm(h, wsvm_ref[...]) + bsvm_ref[...]
    mx = jnp.max(logits, axis=1, keepdims=True)
    shifted = logits - mx
    lse = jnp.log(jnp.sum(jnp.exp(shifted), axis=1, keepdims=True))
    out_ref[...] = shifted - lse
    _refill_step(adj_hbm, buf_ref, sem)


def _gcn_final(adj, y, b, wsvm, bsvm, nclass):
    return pl.pallas_call(
        _final_kernel,
        grid=(M_BLOCKS,),
        in_specs=[
            pl.BlockSpec(memory_space=pltpu.MemorySpace.HBM),
            pl.BlockSpec((N, F), lambda m: (0, 0)),
            pl.BlockSpec((1, F), lambda m: (0, 0)),
            pl.BlockSpec((F, nclass), lambda m: (0, 0)),
            pl.BlockSpec((1, nclass), lambda m: (0, 0)),
        ],
        out_specs=pl.BlockSpec((BM, nclass), lambda m: (m, 0)),
        out_shape=jax.ShapeDtypeStruct((N, nclass), jnp.float32),
        scratch_shapes=[
            pltpu.VMEM((NB, BM, N), jnp.float32),
            pltpu.SemaphoreType.DMA((NB, CC)),
        ],
    )(adj, y, b, wsvm, bsvm)


@jax.jit
def kernel(x, adj, W1, b1, W2, b2, Wsvm, bsvm):
    b1r = b1.reshape(1, F)
    b2r = b2.reshape(1, F)
    bsvmr = bsvm.reshape(1, -1)
    nclass = Wsvm.shape[1]
    h1 = _gcn_layer(adj[0], _feat_mm(x, W1), b1r)
    h2 = _gcn_layer(adj[1], _feat_mm(h1, W2), b2r)
    return _gcn_final(adj[1], _feat_mm(h2, W2), b2r, Wsvm, bsvmr, nclass)


# speedup vs baseline: 2.4779x; 2.4779x over previous
import jax
import jax.numpy as jnp
from jax.experimental import pallas as pl
from jax.experimental.pallas import tpu as pltpu

N = 4096
F = 256
BM = 256
M = N // BM
STEPS = 3 * M
NB = 4
CC = 4
KC = N // CC


def _mm(a, b):
    return jax.lax.dot_general(
        a, b, (((1,), (0,)), ((), ())),
        precision=jax.lax.Precision.DEFAULT,
        preferred_element_type=jnp.float32,
    )


def _block_copy(adj_hbm, buf_ref, sem, g, slot, j):
    layer = jnp.where(g >= M, 1, 0)
    row = jax.lax.rem(g, M) * BM
    return pltpu.make_async_copy(
        adj_hbm.at[layer, pl.ds(row, BM), pl.ds(j * KC, KC)],
        buf_ref.at[slot, :, pl.ds(j * KC, KC)],
        sem.at[slot, j],
    )


def _start_block(adj_hbm, buf_ref, sem, g, slot):
    for j in range(CC):
        _block_copy(adj_hbm, buf_ref, sem, g, slot, j).start()


def _wait_block(adj_hbm, buf_ref, sem, g, slot):
    for j in range(CC):
        _block_copy(adj_hbm, buf_ref, sem, g, slot, j).wait()


def _fused_kernel(adj_hbm, x_hbm, w1_ref, b1_ref, w2_ref, b2_ref,
                  wsvm_ref, bsvm_ref, out_ref,
                  buf_ref, y_ref, h1_ref, h2_ref, xv_ref, sem, xsem):
    g = pl.program_id(0)

    @pl.when(g == 0)
    def _():
        cp_x = pltpu.make_async_copy(x_hbm, xv_ref, xsem)
        cp_x.start()
        for r in range(NB):
            _start_block(adj_hbm, buf_ref, sem, r, r)
        cp_x.wait()
        y_ref[...] = _mm(xv_ref[...], w1_ref[...])

    @pl.when(g == M)
    def _():
        y_ref[...] = _mm(h1_ref[...], w2_ref[...])

    @pl.when(g == 2 * M)
    def _():
        y_ref[...] = _mm(h2_ref[...], w2_ref[...])

    slot = jax.lax.rem(g, NB)
    _wait_block(adj_hbm, buf_ref, sem, g, slot)
    acc = _mm(buf_ref[slot], y_ref[...])

    row = jax.lax.rem(g, M) * BM

    @pl.when(g < M)
    def _():
        h1_ref[pl.ds(row, BM), :] = jnp.maximum(acc + b1_ref[...], 0.0)

    @pl.when(jnp.logical_and(g >= M, g < 2 * M))
    def _():
        h2_ref[pl.ds(row, BM), :] = jnp.maximum(acc + b2_ref[...], 0.0)

    @pl.when(g >= 2 * M)
    def _():
        h = jnp.maximum(acc + b2_ref[...], 0.0)
        logits = _mm(h, wsvm_ref[...]) + bsvm_ref[...]
        mx = jnp.max(logits, axis=1, keepdims=True)
        shifted = logits - mx
        lse = jnp.log(jnp.sum(jnp.exp(shifted), axis=1, keepdims=True))
        out_ref[...] = shifted - lse

    @pl.when(g + NB < STEPS)
    def _():
        _start_block(adj_hbm, buf_ref, sem, g + NB, slot)


@jax.jit
def kernel(x, adj, W1, b1, W2, b2, Wsvm, bsvm):
    nclass = Wsvm.shape[1]
    return pl.pallas_call(
        _fused_kernel,
        grid=(STEPS,),
        in_specs=[
            pl.BlockSpec(memory_space=pltpu.MemorySpace.HBM),
            pl.BlockSpec(memory_space=pltpu.MemorySpace.HBM),
            pl.BlockSpec((F, F), lambda g: (0, 0)),
            pl.BlockSpec((1, F), lambda g: (0, 0)),
            pl.BlockSpec((F, F), lambda g: (0, 0)),
            pl.BlockSpec((1, F), lambda g: (0, 0)),
            pl.BlockSpec((F, nclass), lambda g: (0, 0)),
            pl.BlockSpec((1, nclass), lambda g: (0, 0)),
        ],
        out_specs=pl.BlockSpec((BM, nclass), lambda g: (g % M, 0)),
        out_shape=jax.ShapeDtypeStruct((N, nclass), jnp.float32),
        scratch_shapes=[
            pltpu.VMEM((NB, BM, N), jnp.float32),
            pltpu.VMEM((N, F), jnp.float32),
            pltpu.VMEM((N, F), jnp.float32),
            pltpu.VMEM((N, F), jnp.float32),
            pltpu.VMEM((N, F), jnp.float32),
            pltpu.SemaphoreType.DMA((NB, CC)),
            pltpu.SemaphoreType.DMA,
        ],
    )(adj, x, W1, b1.reshape(1, F), W2, b2.reshape(1, F),
      Wsvm, bsvm.reshape(1, nclass))


# speedup vs baseline: 2.5202x; 1.0171x over previous
import jax
import jax.numpy as jnp
from jax.experimental import pallas as pl
from jax.experimental.pallas import tpu as pltpu

N = 4096
F = 256
BM = 256
M = N // BM
STEPS = 3 * M
DMA_STEPS = 2 * M
NB = 4
CC = 4
KC = N // CC


def _mm(a, b):
    return jax.lax.dot_general(
        a, b, (((1,), (0,)), ((), ())),
        precision=jax.lax.Precision.DEFAULT,
        preferred_element_type=jnp.float32,
    )


def _block_copy(adj_hbm, buf_ref, sem, g, slot, j):
    layer = jnp.where(g >= M, 1, 0)
    row = jax.lax.rem(g, M) * BM
    return pltpu.make_async_copy(
        adj_hbm.at[layer, pl.ds(row, BM), pl.ds(j * KC, KC)],
        buf_ref.at[slot, :, pl.ds(j * KC, KC)],
        sem.at[slot, j],
    )


def _start_block(adj_hbm, buf_ref, sem, g, slot):
    for j in range(CC):
        _block_copy(adj_hbm, buf_ref, sem, g, slot, j).start()


def _wait_block(adj_hbm, buf_ref, sem, g, slot):
    for j in range(CC):
        _block_copy(adj_hbm, buf_ref, sem, g, slot, j).wait()


def _fused_kernel(adj_hbm, x_hbm, w1_ref, b1_ref, w2_ref, b2_ref,
                  wsvm_ref, bsvm_ref, out_ref,
                  buf_ref, y_ref, y3_ref, h1_ref, h2_ref, cache_ref,
                  sem, xsem):
    g = pl.program_id(0)
    slot = jax.lax.rem(g, NB)
    row = jax.lax.rem(g, M) * BM

    @pl.when(g == 0)
    def _():
        cp_x = pltpu.make_async_copy(x_hbm, h2_ref, xsem)
        cp_x.start()
        for r in range(NB):
            _start_block(adj_hbm, buf_ref, sem, r, r)
        cp_x.wait()
        y_ref[...] = _mm(h2_ref[...], w1_ref[...])

    @pl.when(g == M)
    def _():
        y_ref[...] = jax.lax.dot_general(
            h1_ref[...], w2_ref[...].astype(jnp.bfloat16),
            (((1,), (0,)), ((), ())),
            precision=jax.lax.Precision.DEFAULT,
            preferred_element_type=jnp.float32,
        )

    @pl.when(g == 2 * M)
    def _():
        y3_ref[...] = _mm(h2_ref[...], w2_ref[...]).astype(jnp.bfloat16)

    @pl.when(g < DMA_STEPS)
    def _():
        _wait_block(adj_hbm, buf_ref, sem, g, slot)
        acc = _mm(buf_ref[slot], y_ref[...])

        @pl.when(g < M)
        def _():
            h1_ref[pl.ds(row, BM), :] = jnp.maximum(
                acc + b1_ref[...], 0.0).astype(jnp.bfloat16)

        @pl.when(g >= M)
        def _():
            h2_ref[pl.ds(row, BM), :] = jnp.maximum(acc + b2_ref[...], 0.0)
            cache_ref[jax.lax.rem(g, M)] = buf_ref[slot].astype(jnp.bfloat16)

        @pl.when(g + NB < DMA_STEPS)
        def _():
            _start_block(adj_hbm, buf_ref, sem, g + NB, slot)

    @pl.when(g >= DMA_STEPS)
    def _():
        acc = _mm(cache_ref[jax.lax.rem(g, M)], y3_ref[...])
        h = jnp.maximum(acc + b2_ref[...], 0.0)
        logits = _mm(h, wsvm_ref[...]) + bsvm_ref[...]
        mx = jnp.max(logits, axis=1, keepdims=True)
        shifted = logits - mx
        lse = jnp.log(jnp.sum(jnp.exp(shifted), axis=1, keepdims=True))
        out_ref[...] = shifted - lse


@jax.jit
def kernel(x, adj, W1, b1, W2, b2, Wsvm, bsvm):
    nclass = Wsvm.shape[1]
    return pl.pallas_call(
        _fused_kernel,
        grid=(STEPS,),
        in_specs=[
            pl.BlockSpec(memory_space=pltpu.MemorySpace.HBM),
            pl.BlockSpec(memory_space=pltpu.MemorySpace.HBM),
            pl.BlockSpec((F, F), lambda g: (0, 0)),
            pl.BlockSpec((1, F), lambda g: (0, 0)),
            pl.BlockSpec((F, F), lambda g: (0, 0)),
            pl.BlockSpec((1, F), lambda g: (0, 0)),
            pl.BlockSpec((F, nclass), lambda g: (0, 0)),
            pl.BlockSpec((1, nclass), lambda g: (0, 0)),
        ],
        out_specs=pl.BlockSpec((BM, nclass), lambda g: (g % M, 0)),
        out_shape=jax.ShapeDtypeStruct((N, nclass), jnp.float32),
        compiler_params=pltpu.CompilerParams(
            vmem_limit_bytes=64 * 1024 * 1024,
        ),
        scratch_shapes=[
            pltpu.VMEM((NB, BM, N), jnp.float32),
            pltpu.VMEM((N, F), jnp.float32),
            pltpu.VMEM((N, F), jnp.bfloat16),
            pltpu.VMEM((N, F), jnp.bfloat16),
            pltpu.VMEM((N, F), jnp.float32),
            pltpu.VMEM((M, BM, N), jnp.bfloat16),
            pltpu.SemaphoreType.DMA((NB, CC)),
            pltpu.SemaphoreType.DMA,
        ],
    )(adj, x, W1, b1.reshape(1, F), W2, b2.reshape(1, F),
      Wsvm, bsvm.reshape(1, nclass))


# speedup vs baseline: 2.5288x; 1.0034x over previous
import jax
import jax.numpy as jnp
from jax.experimental import pallas as pl
from jax.experimental.pallas import tpu as pltpu

N = 4096
F = 256
BM = 256
M = N // BM
STEPS = 3 * M
DMA_STEPS = 2 * M
NB = 4
CC = 2
KC = N // CC


def _mm(a, b):
    return jax.lax.dot_general(
        a, b, (((1,), (0,)), ((), ())),
        precision=jax.lax.Precision.DEFAULT,
        preferred_element_type=jnp.float32,
    )


def _block_copy(adj_hbm, buf_ref, sem, g, slot, j):
    layer = jnp.where(g >= M, 1, 0)
    row = jax.lax.rem(g, M) * BM
    return pltpu.make_async_copy(
        adj_hbm.at[layer, pl.ds(row, BM), pl.ds(j * KC, KC)],
        buf_ref.at[slot, :, pl.ds(j * KC, KC)],
        sem.at[slot, j],
    )


def _start_block(adj_hbm, buf_ref, sem, g, slot):
    for j in range(CC):
        _block_copy(adj_hbm, buf_ref, sem, g, slot, j).start()


def _wait_block(adj_hbm, buf_ref, sem, g, slot):
    for j in range(CC):
        _block_copy(adj_hbm, buf_ref, sem, g, slot, j).wait()


def _fused_kernel(adj_hbm, x_hbm, w1_ref, b1_ref, w2_ref, b2_ref,
                  wsvm_ref, bsvm_ref, out_ref,
                  buf_ref, y_ref, y3_ref, h1_ref, h2_ref, cache_ref,
                  sem, xsem):
    g = pl.program_id(0)
    slot = jax.lax.rem(g, NB)
    row = jax.lax.rem(g, M) * BM

    @pl.when(g == 0)
    def _():
        cp_x = pltpu.make_async_copy(x_hbm, h2_ref, xsem)
        cp_x.start()
        for r in range(NB):
            _start_block(adj_hbm, buf_ref, sem, r, r)
        cp_x.wait()
        y_ref[...] = _mm(h2_ref[...], w1_ref[...])

    @pl.when(g == M)
    def _():
        y_ref[...] = jax.lax.dot_general(
            h1_ref[...], w2_ref[...].astype(jnp.bfloat16),
            (((1,), (0,)), ((), ())),
            precision=jax.lax.Precision.DEFAULT,
            preferred_element_type=jnp.float32,
        )

    @pl.when(g == 2 * M)
    def _():
        y3_ref[...] = _mm(h2_ref[...], w2_ref[...]).astype(jnp.bfloat16)

    @pl.when(g < DMA_STEPS)
    def _():
        _wait_block(adj_hbm, buf_ref, sem, g, slot)
        acc = _mm(buf_ref[slot], y_ref[...])

        @pl.when(g < M)
        def _():
            h1_ref[pl.ds(row, BM), :] = jnp.maximum(
                acc + b1_ref[...], 0.0).astype(jnp.bfloat16)

        @pl.when(g >= M)
        def _():
            h2_ref[pl.ds(row, BM), :] = jnp.maximum(acc + b2_ref[...], 0.0)
            cache_ref[jax.lax.rem(g, M)] = buf_ref[slot].astype(jnp.bfloat16)

        @pl.when(g + NB < DMA_STEPS)
        def _():
            _start_block(adj_hbm, buf_ref, sem, g + NB, slot)

    @pl.when(g >= DMA_STEPS)
    def _():
        acc = _mm(cache_ref[jax.lax.rem(g, M)], y3_ref[...])
        h = jnp.maximum(acc + b2_ref[...], 0.0)
        logits = _mm(h, wsvm_ref[...]) + bsvm_ref[...]
        mx = jnp.max(logits, axis=1, keepdims=True)
        shifted = logits - mx
        lse = jnp.log(jnp.sum(jnp.exp(shifted), axis=1, keepdims=True))
        out_ref[...] = shifted - lse


@jax.jit
def kernel(x, adj, W1, b1, W2, b2, Wsvm, bsvm):
    nclass = Wsvm.shape[1]
    return pl.pallas_call(
        _fused_kernel,
        grid=(STEPS,),
        in_specs=[
            pl.BlockSpec(memory_space=pltpu.MemorySpace.HBM),
            pl.BlockSpec(memory_space=pltpu.MemorySpace.HBM),
            pl.BlockSpec((F, F), lambda g: (0, 0)),
            pl.BlockSpec((1, F), lambda g: (0, 0)),
            pl.BlockSpec((F, F), lambda g: (0, 0)),
            pl.BlockSpec((1, F), lambda g: (0, 0)),
            pl.BlockSpec((F, nclass), lambda g: (0, 0)),
            pl.BlockSpec((1, nclass), lambda g: (0, 0)),
        ],
        out_specs=pl.BlockSpec((BM, nclass), lambda g: (g % M, 0)),
        out_shape=jax.ShapeDtypeStruct((N, nclass), jnp.float32),
        compiler_params=pltpu.CompilerParams(
            vmem_limit_bytes=64 * 1024 * 1024,
        ),
        scratch_shapes=[
            pltpu.VMEM((NB, BM, N), jnp.float32),
            pltpu.VMEM((N, F), jnp.float32),
            pltpu.VMEM((N, F), jnp.bfloat16),
            pltpu.VMEM((N, F), jnp.bfloat16),
            pltpu.VMEM((N, F), jnp.float32),
            pltpu.VMEM((M, BM, N), jnp.bfloat16),
            pltpu.SemaphoreType.DMA((NB, CC)),
            pltpu.SemaphoreType.DMA,
        ],
    )(adj, x, W1, b1.reshape(1, F), W2, b2.reshape(1, F),
      Wsvm, bsvm.reshape(1, nclass))


# speedup vs baseline: 2.6182x; 1.0353x over previous
import jax
import jax.numpy as jnp
from jax.experimental import pallas as pl
from jax.experimental.pallas import tpu as pltpu

N = 4096
F = 256
BM = 512
M = N // BM
STEPS = 3 * M
DMA_STEPS = 2 * M
NB = 2
CC = 8
KC = N // CC


def _mm(a, b):
    return jax.lax.dot_general(
        a, b, (((1,), (0,)), ((), ())),
        precision=jax.lax.Precision.DEFAULT,
        preferred_element_type=jnp.float32,
    )


def _block_copy(adj_hbm, buf_ref, sem, g, slot, j):
    layer = jnp.where(g >= M, 1, 0)
    row = jax.lax.rem(g, M) * BM
    return pltpu.make_async_copy(
        adj_hbm.at[layer, pl.ds(row, BM), pl.ds(j * KC, KC)],
        buf_ref.at[slot, :, pl.ds(j * KC, KC)],
        sem.at[slot, j],
    )


def _start_block(adj_hbm, buf_ref, sem, g, slot):
    for j in range(CC):
        _block_copy(adj_hbm, buf_ref, sem, g, slot, j).start()


def _wait_block(adj_hbm, buf_ref, sem, g, slot):
    for j in range(CC):
        _block_copy(adj_hbm, buf_ref, sem, g, slot, j).wait()


def _fused_kernel(adj_hbm, x_hbm, w1_ref, b1_ref, w2_ref, b2_ref,
                  wsvm_ref, bsvm_ref, out_ref,
                  buf_ref, y_ref, y3_ref, h1_ref, h2_ref, cache_ref,
                  sem, xsem):
    g = pl.program_id(0)
    slot = jax.lax.rem(g, NB)
    row = jax.lax.rem(g, M) * BM

    @pl.when(g == 0)
    def _():
        cp_x = pltpu.make_async_copy(x_hbm, h2_ref, xsem)
        cp_x.start()
        for r in range(NB):
            _start_block(adj_hbm, buf_ref, sem, r, r)
        cp_x.wait()
        y_ref[...] = _mm(h2_ref[...], w1_ref[...])

    @pl.when(g == M)
    def _():
        y_ref[...] = jax.lax.dot_general(
            h1_ref[...], w2_ref[...].astype(jnp.bfloat16),
            (((1,), (0,)), ((), ())),
            precision=jax.lax.Precision.DEFAULT,
            preferred_element_type=jnp.float32,
        )

    @pl.when(g == 2 * M)
    def _():
        y3_ref[...] = _mm(h2_ref[...], w2_ref[...]).astype(jnp.bfloat16)

    @pl.when(g < DMA_STEPS)
    def _():
        _wait_block(adj_hbm, buf_ref, sem, g, slot)
        acc = _mm(buf_ref[slot], y_ref[...])

        @pl.when(g < M)
        def _():
            h1_ref[pl.ds(row, BM), :] = jnp.maximum(
                acc + b1_ref[...], 0.0).astype(jnp.bfloat16)

        @pl.when(g >= M)
        def _():
            h2_ref[pl.ds(row, BM), :] = jnp.maximum(acc + b2_ref[...], 0.0)
            cache_ref[jax.lax.rem(g, M)] = buf_ref[slot].astype(jnp.bfloat16)

        @pl.when(g + NB < DMA_STEPS)
        def _():
            _start_block(adj_hbm, buf_ref, sem, g + NB, slot)

    @pl.when(g >= DMA_STEPS)
    def _():
        acc = _mm(cache_ref[jax.lax.rem(g, M)], y3_ref[...])
        h = jnp.maximum(acc + b2_ref[...], 0.0)
        logits = _mm(h, wsvm_ref[...]) + bsvm_ref[...]
        mx = jnp.max(logits, axis=1, keepdims=True)
        shifted = logits - mx
        lse = jnp.log(jnp.sum(jnp.exp(shifted), axis=1, keepdims=True))
        out_ref[...] = shifted - lse


@jax.jit
def kernel(x, adj, W1, b1, W2, b2, Wsvm, bsvm):
    nclass = Wsvm.shape[1]
    return pl.pallas_call(
        _fused_kernel,
        grid=(STEPS,),
        in_specs=[
            pl.BlockSpec(memory_space=pltpu.MemorySpace.HBM),
            pl.BlockSpec(memory_space=pltpu.MemorySpace.HBM),
            pl.BlockSpec((F, F), lambda g: (0, 0)),
            pl.BlockSpec((1, F), lambda g: (0, 0)),
            pl.BlockSpec((F, F), lambda g: (0, 0)),
            pl.BlockSpec((1, F), lambda g: (0, 0)),
            pl.BlockSpec((F, nclass), lambda g: (0, 0)),
            pl.BlockSpec((1, nclass), lambda g: (0, 0)),
        ],
        out_specs=pl.BlockSpec((BM, nclass), lambda g: (g % M, 0)),
        out_shape=jax.ShapeDtypeStruct((N, nclass), jnp.float32),
        compiler_params=pltpu.CompilerParams(
            vmem_limit_bytes=64 * 1024 * 1024,
        ),
        scratch_shapes=[
            pltpu.VMEM((NB, BM, N), jnp.float32),
            pltpu.VMEM((N, F), jnp.float32),
            pltpu.VMEM((N, F), jnp.bfloat16),
            pltpu.VMEM((N, F), jnp.bfloat16),
            pltpu.VMEM((N, F), jnp.float32),
            pltpu.VMEM((M, BM, N), jnp.bfloat16),
            pltpu.SemaphoreType.DMA((NB, CC)),
            pltpu.SemaphoreType.DMA,
        ],
    )(adj, x, W1, b1.reshape(1, F), W2, b2.reshape(1, F),
      Wsvm, bsvm.reshape(1, nclass))
